# baseline (speedup 1.0000x reference)
import functools

import jax
import jax.numpy as jnp
from jax import lax
from jax.experimental import pallas as pl
from jax.experimental.pallas import tpu as pltpu
from jax.experimental.pallas import tpu_sc as plsc

VOCAB = 1000000
BATCH = 4096
HIST = 50
D = 64
B = BATCH * HIST
NW = 32
BPW = B // NW
C = 320
NCHUNK = BPW // C

NB = 8192
GRID = (VOCAB + NB - 1) // NB
VLIN = GRID * NB


def _fmt_body(mu_ref, lv_ref, mu_out, lv_out):
    mu = mu_ref[...]
    lv = lv_ref[...]
    mu_out[:, 0:64] = mu[:, : NB // 2].T
    mu_out[:, 64:128] = mu[:, NB // 2 :].T
    lv_out[:, 0:64] = jnp.exp(lv[:, : NB // 2].T)
    lv_out[:, 64:128] = jnp.exp(lv[:, NB // 2 :].T)


def _tc_format(mu_t, lv_t):
    return pl.pallas_call(
        _fmt_body,
        grid=(GRID,),
        in_specs=[
            pl.BlockSpec((64, NB), lambda i: (0, i)),
            pl.BlockSpec((64, NB), lambda i: (0, i)),
        ],
        out_specs=[
            pl.BlockSpec((NB // 2, 128), lambda i: (i, 0)),
            pl.BlockSpec((NB // 2, 128), lambda i: (i, 0)),
        ],
        out_shape=[
            jax.ShapeDtypeStruct((VLIN // 2, 128), jnp.float32),
            jax.ShapeDtypeStruct((VLIN // 2, 128), jnp.float32),
        ],
    )(mu_t, lv_t)


UPW = 50


def _gather_body(ids_hbm, mu_hbm, var_hbm, mu_out, var_out,
                 idx0, idx1, rmu0, rmu1, rlv0, rlv1, tmu0, tmu1, tlv0, tlv1,
                 sg0, sg1, sw0, sw1):
    wid = lax.axis_index("s") * 2 + lax.axis_index("c")
    base_u = wid * UPW
    idx_v = (idx0, idx1)
    r_mu = (rmu0, rmu1)
    r_lv = (rlv0, rlv1)
    t_mu = (tmu0, tmu1)
    t_lv = (tlv0, tlv1)
    sem_g = (sg0, sg1)
    sem_w = (sw0, sw1)
    iota = lax.iota(jnp.int32, 16)

    def issue_gather(u, b):
        off = (base_u + u) * 128
        pltpu.sync_copy(ids_hbm.at[pl.ds(off, 128)], idx_v[b])
        cp_mu = pltpu.async_copy(mu_hbm.at[idx_v[b]], r_mu[b], sem_g[b])
        cp_lv = pltpu.async_copy(var_hbm.at[idx_v[b]], r_lv[b], sem_g[b])
        return (cp_mu, cp_lv)

    lane128 = iota * 128

    def transpose(src, dst):
        @plsc.parallel_loop(0, 128, unroll=2)
        def trow(bl):
            for gg in range(D // 16):
                v = src[bl, pl.ds(gg * 16, 16)]
                plsc.store_scatter(dst, [lane128 + (gg * 2048 + bl)], v)

    prol = [issue_gather(0, 0), issue_gather(1, 1)]

    def body(g, carry):
        for b in (0, 1):
            u = 2 * g + b
            gu = base_u + u
            h = gu // 32
            bb = gu - h * 32

            @pl.when(g > 0)
            def _drain():
                for jb in range(8):
                    pltpu.make_async_copy(mu_out.at[h, jb, bb],
                                          t_mu[b].at[pl.ds(jb * 1024, 1024)],
                                          sem_w[b]).wait()
                    pltpu.make_async_copy(var_out.at[h, jb, bb],
                                          t_lv[b].at[pl.ds(jb * 1024, 1024)],
                                          sem_w[b]).wait()

            for cp in prol[b]:
                cp.wait()
            transpose(r_mu[b], t_mu[b])
            transpose(r_lv[b], t_lv[b])

            @pl.when(u + 2 < UPW)
            def _prefetch():
                issue_gather(u + 2, b)

            for jb in range(8):
                pltpu.async_copy(t_mu[b].at[pl.ds(jb * 1024, 1024)],
                                 mu_out.at[h, jb, bb], sem_w[b])
                pltpu.async_copy(t_lv[b].at[pl.ds(jb * 1024, 1024)],
                                 var_out.at[h, jb, bb], sem_w[b])
        return carry

    lax.fori_loop(0, UPW // 2, body, 0)

    for b in (0, 1):
        for jb in range(8):
            pltpu.make_async_copy(mu_out.at[0, jb, 0],
                                  t_mu[b].at[pl.ds(jb * 1024, 1024)],
                                  sem_w[b]).wait()
            pltpu.make_async_copy(var_out.at[0, jb, 0],
                                  t_lv[b].at[pl.ds(jb * 1024, 1024)],
                                  sem_w[b]).wait()


def _sc_gather(ids_flat, mu_lin, var_lin):
    mesh = plsc.VectorSubcoreMesh(core_axis_name="c", subcore_axis_name="s")
    f = functools.partial(
        pl.kernel,
        mesh=mesh,
        compiler_params=pltpu.CompilerParams(
            use_tc_tiling_on_sc=False, needs_layout_passes=False),
        out_type=(
            jax.ShapeDtypeStruct((HIST, 8, 32, 1024), jnp.float32),
            jax.ShapeDtypeStruct((HIST, 8, 32, 1024), jnp.float32),
        ),
        scratch_types=[
            pltpu.VMEM((128,), jnp.int32),
            pltpu.VMEM((128,), jnp.int32),
            pltpu.VMEM((128, D), jnp.float32),
            pltpu.VMEM((128, D), jnp.float32),
            pltpu.VMEM((128, D), jnp.float32),
            pltpu.VMEM((128, D), jnp.float32),
            pltpu.VMEM((D * 128,), jnp.float32),
            pltpu.VMEM((D * 128,), jnp.float32),
            pltpu.VMEM((D * 128,), jnp.float32),
            pltpu.VMEM((D * 128,), jnp.float32),
            pltpu.SemaphoreType.DMA,
            pltpu.SemaphoreType.DMA,
            pltpu.SemaphoreType.DMA,
            pltpu.SemaphoreType.DMA,
        ],
    )(_gather_body)
    return f(ids_flat, mu_lin, var_lin)


@jax.jit
def _pipeline(token_ids, mu_table, logvar_table):
    mu128, var128 = _tc_format(mu_table.T, logvar_table.T)
    mu_lin = mu128.reshape(VLIN, D)
    var_lin = var128.reshape(VLIN, D)
    ids = token_ids.T.reshape(B).astype(jnp.int32)
    l = ids & (NB - 1)
    ids2 = (ids - l) + jnp.where(l < NB // 2, 2 * l, 2 * l - (NB - 1))
    mu5, var5 = _sc_gather(ids2, mu_lin, var_lin)
    mu5 = mu5.reshape(HIST, 8, 32, 8, 128)
    var5 = var5.reshape(HIST, 8, 32, 8, 128)
    mu = mu5.transpose(2, 4, 0, 1, 3).reshape(BATCH, HIST, D)
    var = var5.transpose(2, 4, 0, 1, 3).reshape(BATCH, HIST, D)
    return (mu, var)


def kernel(token_ids, mu_table, logvar_table):
    return _pipeline(token_ids, mu_table, logvar_table)


# speedup vs baseline: 1.0720x; 1.0720x over previous
import functools

import jax
import jax.numpy as jnp
from jax import lax
from jax.experimental import pallas as pl
from jax.experimental.pallas import tpu as pltpu
from jax.experimental.pallas import tpu_sc as plsc

VOCAB = 1000000
BATCH = 4096
HIST = 50
D = 64
B = BATCH * HIST
NW = 32
BPW = B // NW
C = 320
NCHUNK = BPW // C

NB = 8192
GRID = (VOCAB + NB - 1) // NB
VLIN = GRID * NB


def _fmt_body(mu_ref, lv_ref, mu_out, lv_out):
    mu = mu_ref[...]
    lv = lv_ref[...]
    mu_out[:, 0:64] = mu[:, : NB // 2].T
    mu_out[:, 64:128] = mu[:, NB // 2 :].T
    lv_out[:, 0:64] = jnp.exp(lv[:, : NB // 2].T)
    lv_out[:, 64:128] = jnp.exp(lv[:, NB // 2 :].T)


def _tc_format(mu_t, lv_t):
    return pl.pallas_call(
        _fmt_body,
        grid=(GRID,),
        in_specs=[
            pl.BlockSpec((64, NB), lambda i: (0, i)),
            pl.BlockSpec((64, NB), lambda i: (0, i)),
        ],
        out_specs=[
            pl.BlockSpec((NB // 2, 128), lambda i: (i, 0)),
            pl.BlockSpec((NB // 2, 128), lambda i: (i, 0)),
        ],
        out_shape=[
            jax.ShapeDtypeStruct((VLIN // 2, 128), jnp.float32),
            jax.ShapeDtypeStruct((VLIN // 2, 128), jnp.float32),
        ],
    )(mu_t, lv_t)


UPW = 50


def _gather_body(ids_hbm, mu_hbm, var_hbm, mu_out, var_out,
                 idx0, idx1, rmu0, rmu1, rlv0, rlv1, tmu0, tmu1, tlv0, tlv1,
                 sg0, sg1, sw0, sw1):
    wid = lax.axis_index("s") * 2 + lax.axis_index("c")
    base_u = wid * UPW
    idx_v = (idx0, idx1)
    r_mu = (rmu0, rmu1)
    r_lv = (rlv0, rlv1)
    t_mu = (tmu0, tmu1)
    t_lv = (tlv0, tlv1)
    sem_g = (sg0, sg1)
    sem_w = (sw0, sw1)
    iota = lax.iota(jnp.int32, 16)

    def issue_gather(u, b):
        off = (base_u + u) * 128
        pltpu.sync_copy(ids_hbm.at[pl.ds(off, 128)], idx_v[b])
        cp_mu = pltpu.async_copy(mu_hbm.at[idx_v[b]], r_mu[b], sem_g[b])
        cp_lv = pltpu.async_copy(var_hbm.at[idx_v[b]], r_lv[b], sem_g[b])
        return (cp_mu, cp_lv)

    iota128 = iota * 128
    skew = [(iota + sk) & 15 for sk in range(16)]
    dvec = [iota128 + skew[sk] for sk in range(16)]
    cvec = [gg * 16 + iota for gg in range(D // 16)]

    def transpose(src, dst):
        @plsc.parallel_loop(0, 8, unroll=1)
        def tblk(bi):
            bl0 = bi * 16
            for gg in range(D // 16):
                for sk in range(16):
                    v = plsc.load_gather(src, [bl0 + skew[sk], cvec[gg]])
                    plsc.store_scatter(dst, [dvec[sk] + (gg * 2048 + bl0)], v)

    prol = [issue_gather(0, 0), issue_gather(1, 1)]

    def body(g, carry):
        for b in (0, 1):
            u = 2 * g + b
            gu = base_u + u
            h = gu // 32
            bb = gu - h * 32

            @pl.when(g > 0)
            def _drain():
                for jb in range(8):
                    pltpu.make_async_copy(mu_out.at[h, jb, bb],
                                          t_mu[b].at[pl.ds(jb * 1024, 1024)],
                                          sem_w[b]).wait()
                    pltpu.make_async_copy(var_out.at[h, jb, bb],
                                          t_lv[b].at[pl.ds(jb * 1024, 1024)],
                                          sem_w[b]).wait()

            for cp in prol[b]:
                cp.wait()
            transpose(r_mu[b], t_mu[b])
            transpose(r_lv[b], t_lv[b])

            @pl.when(u + 2 < UPW)
            def _prefetch():
                issue_gather(u + 2, b)

            for jb in range(8):
                pltpu.async_copy(t_mu[b].at[pl.ds(jb * 1024, 1024)],
                                 mu_out.at[h, jb, bb], sem_w[b])
                pltpu.async_copy(t_lv[b].at[pl.ds(jb * 1024, 1024)],
                                 var_out.at[h, jb, bb], sem_w[b])
        return carry

    lax.fori_loop(0, UPW // 2, body, 0)

    for b in (0, 1):
        for jb in range(8):
            pltpu.make_async_copy(mu_out.at[0, jb, 0],
                                  t_mu[b].at[pl.ds(jb * 1024, 1024)],
                                  sem_w[b]).wait()
            pltpu.make_async_copy(var_out.at[0, jb, 0],
                                  t_lv[b].at[pl.ds(jb * 1024, 1024)],
                                  sem_w[b]).wait()


def _sc_gather(ids_flat, mu_lin, var_lin):
    mesh = plsc.VectorSubcoreMesh(core_axis_name="c", subcore_axis_name="s")
    f = functools.partial(
        pl.kernel,
        mesh=mesh,
        compiler_params=pltpu.CompilerParams(
            use_tc_tiling_on_sc=False, needs_layout_passes=False),
        out_type=(
            jax.ShapeDtypeStruct((HIST, 8, 32, 1024), jnp.float32),
            jax.ShapeDtypeStruct((HIST, 8, 32, 1024), jnp.float32),
        ),
        scratch_types=[
            pltpu.VMEM((128,), jnp.int32),
            pltpu.VMEM((128,), jnp.int32),
            pltpu.VMEM((128, D), jnp.float32),
            pltpu.VMEM((128, D), jnp.float32),
            pltpu.VMEM((128, D), jnp.float32),
            pltpu.VMEM((128, D), jnp.float32),
            pltpu.VMEM((D * 128,), jnp.float32),
            pltpu.VMEM((D * 128,), jnp.float32),
            pltpu.VMEM((D * 128,), jnp.float32),
            pltpu.VMEM((D * 128,), jnp.float32),
            pltpu.SemaphoreType.DMA,
            pltpu.SemaphoreType.DMA,
            pltpu.SemaphoreType.DMA,
            pltpu.SemaphoreType.DMA,
        ],
    )(_gather_body)
    return f(ids_flat, mu_lin, var_lin)


@jax.jit
def _pipeline(token_ids, mu_table, logvar_table):
    mu128, var128 = _tc_format(mu_table.T, logvar_table.T)
    mu_lin = mu128.reshape(VLIN, D)
    var_lin = var128.reshape(VLIN, D)
    ids = token_ids.T.reshape(B).astype(jnp.int32)
    l = ids & (NB - 1)
    ids2 = (ids - l) + jnp.where(l < NB // 2, 2 * l, 2 * l - (NB - 1))
    mu5, var5 = _sc_gather(ids2, mu_lin, var_lin)
    mu5 = mu5.reshape(HIST, 8, 32, 8, 128)
    var5 = var5.reshape(HIST, 8, 32, 8, 128)
    mu = mu5.transpose(2, 4, 0, 1, 3).reshape(BATCH, HIST, D)
    var = var5.transpose(2, 4, 0, 1, 3).reshape(BATCH, HIST, D)
    return (mu, var)


def kernel(token_ids, mu_table, logvar_table):
    return _pipeline(token_ids, mu_table, logvar_table)


# speedup vs baseline: 1.1336x; 1.0575x over previous
import functools

import jax
import jax.numpy as jnp
from jax import lax
from jax.experimental import pallas as pl
from jax.experimental.pallas import tpu as pltpu
from jax.experimental.pallas import tpu_sc as plsc

VOCAB = 1000000
BATCH = 4096
HIST = 50
D = 64
B = BATCH * HIST
NW = 32
BPW = B // NW
C = 320
NCHUNK = BPW // C

NB = 16384
GRID = (VOCAB + NB - 1) // NB
VLIN = GRID * NB


def _fmt_body(mu_ref, lv_ref, mu_out, lv_out):
    mu = mu_ref[...]
    lv = lv_ref[...]
    mu_out[:, 0:64] = mu[:, : NB // 2].T
    mu_out[:, 64:128] = mu[:, NB // 2 :].T
    lv_out[:, 0:64] = jnp.exp(lv[:, : NB // 2].T)
    lv_out[:, 64:128] = jnp.exp(lv[:, NB // 2 :].T)


def _tc_format(mu_t, lv_t):
    return pl.pallas_call(
        _fmt_body,
        grid=(GRID,),
        in_specs=[
            pl.BlockSpec((64, NB), lambda i: (0, i)),
            pl.BlockSpec((64, NB), lambda i: (0, i)),
        ],
        out_specs=[
            pl.BlockSpec((NB // 2, 128), lambda i: (i, 0)),
            pl.BlockSpec((NB // 2, 128), lambda i: (i, 0)),
        ],
        out_shape=[
            jax.ShapeDtypeStruct((VLIN // 2, 128), jnp.float32),
            jax.ShapeDtypeStruct((VLIN // 2, 128), jnp.float32),
        ],
    )(mu_t, lv_t)


def _gather_body(ids_hbm, mu_hbm, var_hbm, mu_out, var_out,
                 idx0, idx1, mu0, mu1, lv0, lv1,
                 sg0, sg1, sw0, sw1):
    wid = lax.axis_index("s") * 2 + lax.axis_index("c")
    base = wid * BPW
    idx_v = (idx0, idx1)
    mu_v = (mu0, mu1)
    lv_v = (lv0, lv1)
    sem_g = (sg0, sg1)
    sem_w = (sw0, sw1)

    pend_g = [None, None]
    pend_w = [None, None]

    def issue_gather(ci, b):
        off = base + ci * C
        pltpu.sync_copy(ids_hbm.at[pl.ds(off, C)], idx_v[b])
        cp_mu = pltpu.async_copy(mu_hbm.at[idx_v[b]], mu_v[b], sem_g[b])
        cp_lv = pltpu.async_copy(var_hbm.at[idx_v[b]], lv_v[b], sem_g[b])
        pend_g[b] = (cp_mu, cp_lv)

    issue_gather(0, 0)
    for ci in range(NCHUNK):
        b = ci & 1
        nb = 1 - b
        if ci + 1 < NCHUNK:
            if pend_w[nb] is not None:
                for cp in pend_w[nb]:
                    cp.wait()
            issue_gather(ci + 1, nb)
        off = base + ci * C
        cp_mu, cp_lv = pend_g[b]
        cp_mu.wait()
        w_mu = pltpu.async_copy(mu_v[b], mu_out.at[pl.ds(off, C)], sem_w[b])
        cp_lv.wait()
        w_lv = pltpu.async_copy(lv_v[b], var_out.at[pl.ds(off, C)], sem_w[b])
        pend_w[b] = (w_mu, w_lv)

    for b in (0, 1):
        for cp in pend_w[b]:
            cp.wait()


def _sc_gather(ids_flat, mu_lin, var_lin):
    mesh = plsc.VectorSubcoreMesh(core_axis_name="c", subcore_axis_name="s")
    f = functools.partial(
        pl.kernel,
        mesh=mesh,
        compiler_params=pltpu.CompilerParams(use_tc_tiling_on_sc=False),
        out_type=(
            jax.ShapeDtypeStruct((B, D), jnp.float32),
            jax.ShapeDtypeStruct((B, D), jnp.float32),
        ),
        scratch_types=[
            pltpu.VMEM((C,), jnp.int32),
            pltpu.VMEM((C,), jnp.int32),
            pltpu.VMEM((C, D), jnp.float32),
            pltpu.VMEM((C, D), jnp.float32),
            pltpu.VMEM((C, D), jnp.float32),
            pltpu.VMEM((C, D), jnp.float32),
            pltpu.SemaphoreType.DMA,
            pltpu.SemaphoreType.DMA,
            pltpu.SemaphoreType.DMA,
            pltpu.SemaphoreType.DMA,
        ],
    )(_gather_body)
    return f(ids_flat, mu_lin, var_lin)


@jax.jit
def _pipeline(token_ids, mu_table, logvar_table):
    mu128, var128 = _tc_format(mu_table.T, logvar_table.T)
    mu_lin = mu128.reshape(VLIN, D)
    var_lin = var128.reshape(VLIN, D)
    ids = token_ids.T.reshape(B).astype(jnp.int32)
    l = ids & (NB - 1)
    ids2 = (ids - l) + jnp.where(l < NB // 2, 2 * l, 2 * l - (NB - 1))
    mu, var = _sc_gather(ids2, mu_lin, var_lin)
    mu = mu.reshape(HIST, BATCH, D).transpose(1, 0, 2)
    var = var.reshape(HIST, BATCH, D).transpose(1, 0, 2)
    return (mu, var)


def kernel(token_ids, mu_table, logvar_table):
    return _pipeline(token_ids, mu_table, logvar_table)
